# baseline (speedup 1.0000x reference)
import functools

import jax
import jax.numpy as jnp
import numpy as np
from jax import lax
from jax.experimental import pallas as pl
from jax.experimental.pallas import tpu as pltpu
from jax.experimental.pallas import tpu_sc as plsc

N = 10000
E = 320000
F = 128
GN = 64
NB = 64

NC = 2
NS = 16
NW = NC * NS
CHUNK = 128
EPT = 10240
NCHUNK = EPT // CHUNK
TOTCHUNK = NW * NCHUNK
C0CH = 110
C1CH = 50
EPAD = NW * EPT
NPAD = 10240
BN = 512
NBLK = NPAD // BN
RPT = 640
LASTR = N - (NS - 1) * RPT
BE = 1024


def _softsign(v):
    return v / (1.0 + jnp.abs(v))


def _pack_perm():
    p = np.zeros((F, F), np.float32)
    for g in range(F // 32):
        for k in range(16):
            p[32 * g + k, 32 * g + 2 * k] = 1.0
            p[32 * g + 16 + k, 32 * g + 2 * k + 1] = 1.0
    return p


_PERM = _pack_perm()


def _edge_body(ea_ref, tc_ref, tn_ref, perm,
               we1c, be1c, we2c, be2c,
               we1n, be1n, we2n, be2n,
               mc_ref, mn_ref):
    ea = ea_ref[:]
    eacol = ea[:, None]
    hc = _softsign(eacol * we1c[:] + be1c[:])
    mc = _softsign(jnp.dot(hc, we2c[:], preferred_element_type=jnp.float32)
                   + be2c[:])
    hn = _softsign(eacol * we1n[:] + be1n[:])
    mn = _softsign(jnp.dot(hn, we2n[:], preferred_element_type=jnp.float32)
                   + be2n[:])
    maskc = (ea <= tc_ref[0]).astype(jnp.float32)[:, None]
    maskn = (ea <= tn_ref[0]).astype(jnp.float32)[:, None]
    mc_ref[:] = jnp.dot(mc * maskc, perm[:],
                        preferred_element_type=jnp.float32
                        ).astype(jnp.bfloat16)
    mn_ref[:] = jnp.dot(mn * maskn, perm[:],
                        preferred_element_type=jnp.float32
                        ).astype(jnp.bfloat16)


def _edge_weights(ea_pad, t_cov, t_ncov, perm,
                  We1_c, be1_c, We2_c, be2_c,
                  We1_n, be1_n, We2_n, be2_n):
    full = lambda a: pl.BlockSpec(a.shape, lambda i: (0,) * a.ndim)
    smem = pl.BlockSpec(memory_space=pltpu.SMEM)
    return pl.pallas_call(
        _edge_body,
        grid=(EPAD // BE,),
        in_specs=[
            pl.BlockSpec((BE,), lambda i: (i,)),
            smem, smem, full(perm),
            full(We1_c), full(be1_c), full(We2_c), full(be2_c),
            full(We1_n), full(be1_n), full(We2_n), full(be2_n),
        ],
        out_specs=[
            pl.BlockSpec((BE, F), lambda i: (i, 0)),
            pl.BlockSpec((BE, F), lambda i: (i, 0)),
        ],
        out_shape=[
            jax.ShapeDtypeStruct((EPAD, F), jnp.bfloat16),
            jax.ShapeDtypeStruct((EPAD, F), jnp.bfloat16),
        ],
    )(ea_pad, t_cov, t_ncov, perm,
      We1_c, be1_c, We2_c, be2_c,
      We1_n, be1_n, We2_n, be2_n)


def _sc_agg_body(x_hbm, idx_hbm, m_hbm, zeros_hbm, parts_hbm,
                 idx_v, xs_v, xd_v, mm_v, acc,
                 gx0, gd0, sf0, sb0, gm0, gm1):
    c = lax.axis_index("c")
    s = lax.axis_index("s")

    @pl.loop(0, CHUNK)
    def _zrow(e):
        for v in range(F // 16):
            xs_v[e, pl.ds(v * 16, 16)] = jnp.zeros((16,), jnp.float32)

    @pl.when(s < NS - 1)
    def _():
        for k in range(RPT // CHUNK):
            pltpu.sync_copy(xs_v, acc.at[pl.ds(s * RPT + k * CHUNK, CHUNK)])

    @pl.when(s == NS - 1)
    def _():
        base = (NS - 1) * RPT
        for k in range(LASTR // CHUNK):
            pltpu.sync_copy(xs_v, acc.at[pl.ds(base + k * CHUNK, CHUNK)])
        rem = LASTR % CHUNK
        if rem:
            pltpu.sync_copy(xs_v.at[pl.ds(0, rem)],
                            acc.at[pl.ds(base + LASTR - rem, rem)])

    plsc.subcore_barrier()

    cbase = jnp.where(c == 0, s * C0CH, NS * C0CH + s * C1CH)
    ccnt = jnp.where(c == 0, C0CH, C1CH)
    @pl.loop(0, ccnt)
    def _chunk(jj):
        j = cbase + jj
        pltpu.sync_copy(idx_hbm.at[j], idx_v.at[0])
        d0 = pltpu.async_copy(x_hbm.at[idx_v.at[0].at[0]], xs_v, gx0)
        d1 = pltpu.async_copy(x_hbm.at[idx_v.at[0].at[1]], xd_v, gd0)
        pltpu.sync_copy(m_hbm.at[j], mm_v.at[0])
        d0.wait()
        d1.wait()

        @pl.loop(0, CHUNK)
        def _row(e):
            for v in range(F // 32):
                mi = mm_v[0, pl.ds(e * (F // 2) + 16 * v, 16)]
                ma = lax.bitcast_convert_type(mi << 16, jnp.float32)
                mb2 = lax.bitcast_convert_type(mi & jnp.int32(-65536),
                                               jnp.float32)
                sl0 = pl.ds(32 * v, 16)
                sl1 = pl.ds(32 * v + 16, 16)
                xs_v[e, sl0] = xs_v[e, sl0] * ma
                xs_v[e, sl1] = xs_v[e, sl1] * mb2
                xd_v[e, sl0] = xd_v[e, sl0] * ma
                xd_v[e, sl1] = xd_v[e, sl1] * mb2

        d2 = pltpu.async_copy(xs_v, acc.at[idx_v.at[0].at[1]],
                              sf0, add=True)
        d3 = pltpu.async_copy(xd_v, acc.at[idx_v.at[0].at[0]],
                              sb0, add=True)
        d2.wait()
        d3.wait()

    plsc.subcore_barrier()

    @pl.when(s < NS - 1)
    def _():
        pltpu.sync_copy(acc.at[pl.ds(s * RPT, RPT)],
                        parts_hbm.at[c].at[pl.ds(s * RPT, RPT)])

    @pl.when(s == NS - 1)
    def _():
        pltpu.sync_copy(acc.at[pl.ds((NS - 1) * RPT, LASTR)],
                        parts_hbm.at[c].at[pl.ds((NS - 1) * RPT, LASTR)])
        pltpu.sync_copy(zeros_hbm.at[pl.ds(0, NPAD - N)],
                        parts_hbm.at[c].at[pl.ds(N, NPAD - N)])


@functools.cache
def _build_sc_kernel():
    return pl.kernel(
        _sc_agg_body,
        out_type=jax.ShapeDtypeStruct((NC, NPAD, F), jnp.float32),
        mesh=plsc.VectorSubcoreMesh(core_axis_name="c", subcore_axis_name="s",
                                    num_cores=NC, num_subcores=NS),
        scratch_types=(
            [
                pltpu.VMEM((2, 2, CHUNK), jnp.int32),
                pltpu.VMEM((CHUNK, F), jnp.float32),
                pltpu.VMEM((CHUNK, F), jnp.float32),
                pltpu.VMEM((2, CHUNK * F // 2), jnp.int32),
                pltpu.VMEM_SHARED((N, F), jnp.float32),
            ]
            + [pltpu.SemaphoreType.DMA] * 6
        ),
    )


def _sc_aggregate(x_pad, idx, m, zeros_hbm):
    return _build_sc_kernel()(x_pad, idx, m, zeros_hbm)


def _node_body(x_ref, a0_ref, a1_ref, t_ref,
               we1, be1, we2, be2,
               wroot, broot, wi1a, wi1b, bi1, wi2, bi2, wj, bj,
               out_ref):
    x = x_ref[:]
    hl = _softsign(we1[:] + be1[:])
    ml = _softsign(jnp.dot(hl, we2[:], preferred_element_type=jnp.float32)
                   + be2[:])
    ml = ml * jnp.where(t_ref[0] >= 1.0, 1.0, 0.0)
    h1 = (a0_ref[0] + a1_ref[0] + 2.0 * x * ml
          + jnp.dot(x, wroot[:], preferred_element_type=jnp.float32)
          + broot[:])
    a = _softsign(jnp.dot(h1, wi1a[:], preferred_element_type=jnp.float32)
                  + jnp.dot(x, wi1b[:], preferred_element_type=jnp.float32)
                  + bi1[:])
    a = _softsign(jnp.dot(a, wi2[:], preferred_element_type=jnp.float32)
                  + bi2[:])
    sm = jax.nn.softmax(a, axis=1)
    out_ref[:] = sm * _softsign(
        jnp.dot(x, wj[:], preferred_element_type=jnp.float32) + bj[:])


def _node_update(x_pad, parts, t,
                 We1, be1, We2, be2,
                 Wroot, broot, Wi1, bi1, Wi2, bi2, Wj, bj, width):
    full = lambda a: pl.BlockSpec(a.shape, lambda i: (0,) * a.ndim)
    smem = pl.BlockSpec(memory_space=pltpu.SMEM)
    blk = lambda w: pl.BlockSpec((BN, w), lambda i: (i, 0))
    wi1a = Wi1[:F]
    wi1b = Wi1[F:]
    return pl.pallas_call(
        _node_body,
        grid=(NBLK,),
        in_specs=[
            blk(F),
            pl.BlockSpec((1, BN, F), lambda i: (0, i, 0)),
            pl.BlockSpec((1, BN, F), lambda i: (1, i, 0)),
            smem,
            full(We1), full(be1), full(We2), full(be2),
            full(Wroot), full(broot), full(wi1a), full(wi1b), full(bi1),
            full(Wi2), full(bi2), full(Wj), full(bj),
        ],
        out_specs=blk(width),
        out_shape=jax.ShapeDtypeStruct((NPAD, width), jnp.float32),
    )(x_pad, parts, parts, t,
      We1, be1, We2, be2,
      Wroot, broot, wi1a, wi1b, bi1, Wi2, bi2, Wj, bj)


def _pool_body(x_ref, h_ref, b_ref,
               wf0, bf0, wf1, bf1, wf2, bf2,
               out_ref, pool):
    i = pl.program_id(0)

    @pl.when(i == 0)
    def _():
        pool[:] = jnp.zeros_like(pool)

    lig = jnp.where(x_ref[:, 14:15] == -1.0, 0.0, h_ref[:])
    seg = b_ref[:][:, None]
    oh = (seg == lax.broadcasted_iota(jnp.int32, (BN, NB), 1)
          ).astype(jnp.float32)
    pool[:] += lax.dot_general(oh, lig, (((0,), (0,)), ((), ())),
                               preferred_element_type=jnp.float32)

    @pl.when(i == NBLK - 1)
    def _():
        h = jax.nn.relu(jnp.dot(pool[:], wf0[:],
                                preferred_element_type=jnp.float32) + bf0[:])
        h = jax.nn.relu(jnp.dot(h, wf1[:],
                                preferred_element_type=jnp.float32) + bf1[:])
        out_ref[:] = jnp.dot(h, wf2[:],
                             preferred_element_type=jnp.float32) + bf2[:]


def _pool_mlp(x_pad, ncov, batch_pad, Wf0, bf0, Wf1, bf1, Wf2, bf2):
    full = lambda a: pl.BlockSpec(a.shape, lambda i: (0,) * a.ndim)
    return pl.pallas_call(
        _pool_body,
        grid=(NBLK,),
        in_specs=[
            pl.BlockSpec((BN, F), lambda i: (i, 0)),
            pl.BlockSpec((BN, GN), lambda i: (i, 0)),
            pl.BlockSpec((BN,), lambda i: (i,)),
            full(Wf0), full(bf0), full(Wf1), full(bf1), full(Wf2), full(bf2),
        ],
        out_specs=pl.BlockSpec((NB, 1), lambda i: (0, 0)),
        out_shape=jax.ShapeDtypeStruct((NB, 1), jnp.float32),
        scratch_shapes=[pltpu.VMEM((NB, NB), jnp.float32)],
    )(x_pad, ncov, batch_pad, Wf0, bf0, Wf1, bf1, Wf2, bf2)


def kernel(x, edge_index, edge_attr, batch, t_cov, t_ncov,
           We1_c, be1_c, We2_c, be2_c, Wroot_c, broot_c,
           Wi1_c, bi1_c, Wi2_c, bi2_c, Wj_c, bj_c,
           We1_n, be1_n, We2_n, be2_n, Wroot_n, broot_n,
           Wi1_n, bi1_n, Wi2_n, bi2_n, Wj_n, bj_n,
           Wf0, bf0, Wf1, bf1, Wf2, bf2):
    epad = EPAD - E
    ea_pad = jnp.concatenate([edge_attr,
                              jnp.full((epad,), 2.0, jnp.float32)])
    src_pad = jnp.concatenate([edge_index[0],
                               jnp.zeros((epad,), jnp.int32)])
    dst_pad = jnp.concatenate([edge_index[1],
                               jnp.zeros((epad,), jnp.int32)])
    x_pad = jnp.concatenate([x, jnp.zeros((NPAD - N, F), jnp.float32)])
    batch_pad = jnp.concatenate([batch,
                                 jnp.full((NPAD - N,), NB, jnp.int32)])
    zeros_hbm = jnp.zeros((RPT, F), jnp.float32)

    m_c, m_n = _edge_weights(ea_pad, t_cov, t_ncov, jnp.asarray(_PERM),
                             We1_c, be1_c, We2_c, be2_c,
                             We1_n, be1_n, We2_n, be2_n)
    idx = jnp.stack([src_pad.reshape(TOTCHUNK, CHUNK),
                     dst_pad.reshape(TOTCHUNK, CHUNK)], axis=1)
    m_c = lax.bitcast_convert_type(
        m_c.reshape(TOTCHUNK, CHUNK, F // 2, 2), jnp.int32
    ).reshape(TOTCHUNK, CHUNK * F // 2)
    m_n = lax.bitcast_convert_type(
        m_n.reshape(TOTCHUNK, CHUNK, F // 2, 2), jnp.int32
    ).reshape(TOTCHUNK, CHUNK * F // 2)

    parts_c = _sc_aggregate(x_pad, idx, m_c, zeros_hbm)
    cov = _node_update(x_pad, parts_c, t_cov,
                       We1_c, be1_c, We2_c, be2_c,
                       Wroot_c, broot_c, Wi1_c, bi1_c,
                       Wi2_c, bi2_c, Wj_c, bj_c, F)

    parts_n = _sc_aggregate(cov, idx, m_n, zeros_hbm)
    ncov = _node_update(cov, parts_n, t_ncov,
                        We1_n, be1_n, We2_n, be2_n,
                        Wroot_n, broot_n, Wi1_n, bi1_n,
                        Wi2_n, bi2_n, Wj_n, bj_n, GN)

    return _pool_mlp(x_pad, ncov, batch_pad, Wf0, bf0, Wf1, bf1, Wf2, bf2)


# speedup vs baseline: 1.0424x; 1.0424x over previous
import functools

import jax
import jax.numpy as jnp
import numpy as np
from jax import lax
from jax.experimental import pallas as pl
from jax.experimental.pallas import tpu as pltpu
from jax.experimental.pallas import tpu_sc as plsc

N = 10000
E = 320000
F = 128
GN = 64
NB = 64

NC = 2
NS = 16
NW = NC * NS
CHUNK = 128
EPT = 10240
NCHUNK = EPT // CHUNK
TOTCHUNK = NW * NCHUNK
C0CH = 110
C1CH = 50
EPAD = NW * EPT
NPAD = 10240
BN = 512
NBLK = NPAD // BN
RPT = 640
LASTR = N - (NS - 1) * RPT
BE = 1024


def _softsign(v):
    return v / (1.0 + jnp.abs(v))


def _pack_perm():
    p = np.zeros((F, F), np.float32)
    for g in range(F // 32):
        for k in range(16):
            p[32 * g + k, 32 * g + 2 * k] = 1.0
            p[32 * g + 16 + k, 32 * g + 2 * k + 1] = 1.0
    return p


_PERM = _pack_perm()


def _edge_body(ea_ref, tc_ref, tn_ref, perm,
               we1c, be1c, we2c, be2c,
               we1n, be1n, we2n, be2n,
               mc_ref, mn_ref):
    ea = ea_ref[:]
    eacol = ea[:, None]
    hc = _softsign(eacol * we1c[:] + be1c[:])
    mc = _softsign(jnp.dot(hc, we2c[:], preferred_element_type=jnp.float32)
                   + be2c[:])
    hn = _softsign(eacol * we1n[:] + be1n[:])
    mn = _softsign(jnp.dot(hn, we2n[:], preferred_element_type=jnp.float32)
                   + be2n[:])
    maskc = (ea <= tc_ref[0]).astype(jnp.float32)[:, None]
    maskn = (ea <= tn_ref[0]).astype(jnp.float32)[:, None]
    mc_ref[:] = jnp.dot(mc * maskc, perm[:],
                        preferred_element_type=jnp.float32
                        ).astype(jnp.bfloat16)
    mn_ref[:] = jnp.dot(mn * maskn, perm[:],
                        preferred_element_type=jnp.float32
                        ).astype(jnp.bfloat16)


def _edge_weights(ea_pad, t_cov, t_ncov, perm,
                  We1_c, be1_c, We2_c, be2_c,
                  We1_n, be1_n, We2_n, be2_n):
    full = lambda a: pl.BlockSpec(a.shape, lambda i: (0,) * a.ndim)
    smem = pl.BlockSpec(memory_space=pltpu.SMEM)
    return pl.pallas_call(
        _edge_body,
        grid=(EPAD // BE,),
        in_specs=[
            pl.BlockSpec((BE,), lambda i: (i,)),
            smem, smem, full(perm),
            full(We1_c), full(be1_c), full(We2_c), full(be2_c),
            full(We1_n), full(be1_n), full(We2_n), full(be2_n),
        ],
        out_specs=[
            pl.BlockSpec((BE, F), lambda i: (i, 0)),
            pl.BlockSpec((BE, F), lambda i: (i, 0)),
        ],
        out_shape=[
            jax.ShapeDtypeStruct((EPAD, F), jnp.bfloat16),
            jax.ShapeDtypeStruct((EPAD, F), jnp.bfloat16),
        ],
    )(ea_pad, t_cov, t_ncov, perm,
      We1_c, be1_c, We2_c, be2_c,
      We1_n, be1_n, We2_n, be2_n)


def _sc_agg_body(x_hbm, idx_hbm, m_hbm, zeros_hbm, parts_hbm,
                 idx_v, xs_v, xd_v, mm_v, acc,
                 gx0, gd0, sf0, sb0, gm0, gm1):
    c = lax.axis_index("c")
    s = lax.axis_index("s")

    @pl.loop(0, CHUNK)
    def _zrow(e):
        for v in range(F // 16):
            xs_v[e, pl.ds(v * 16, 16)] = jnp.zeros((16,), jnp.float32)

    @pl.when(s < NS - 1)
    def _():
        for k in range(RPT // CHUNK):
            pltpu.sync_copy(xs_v, acc.at[pl.ds(s * RPT + k * CHUNK, CHUNK)])

    @pl.when(s == NS - 1)
    def _():
        base = (NS - 1) * RPT
        for k in range(LASTR // CHUNK):
            pltpu.sync_copy(xs_v, acc.at[pl.ds(base + k * CHUNK, CHUNK)])
        rem = LASTR % CHUNK
        if rem:
            pltpu.sync_copy(xs_v.at[pl.ds(0, rem)],
                            acc.at[pl.ds(base + LASTR - rem, rem)])

    plsc.subcore_barrier()

    cbase = jnp.where(c == 0, s * C0CH, NS * C0CH + s * C1CH)
    ccnt = jnp.where(c == 0, C0CH, C1CH)
    @pl.loop(0, ccnt)
    def _chunk(jj):
        j = cbase + jj
        pltpu.sync_copy(idx_hbm.at[j], idx_v.at[0])
        d0 = pltpu.async_copy(x_hbm.at[idx_v.at[0].at[0]], xs_v, gx0)
        d1 = pltpu.async_copy(x_hbm.at[idx_v.at[0].at[1]], xd_v, gd0)
        pltpu.sync_copy(m_hbm.at[j], mm_v)
        d0.wait()
        d1.wait()

        @pl.loop(0, CHUNK // 2)
        def _rowpair(r):
            for h in range(2):
                e = 2 * r + h
                for v in range(F // 32):
                    mi = mm_v[r, pl.ds(h * (F // 2) + 16 * v, 16)]
                    ma = lax.bitcast_convert_type(mi << 16, jnp.float32)
                    mb2 = lax.bitcast_convert_type(mi & jnp.int32(-65536),
                                                   jnp.float32)
                    sl0 = pl.ds(32 * v, 16)
                    sl1 = pl.ds(32 * v + 16, 16)
                    xs_v[e, sl0] = xs_v[e, sl0] * ma
                    xs_v[e, sl1] = xs_v[e, sl1] * mb2
                    xd_v[e, sl0] = xd_v[e, sl0] * ma
                    xd_v[e, sl1] = xd_v[e, sl1] * mb2

        d2 = pltpu.async_copy(xs_v, acc.at[idx_v.at[0].at[1]],
                              sf0, add=True)
        d3 = pltpu.async_copy(xd_v, acc.at[idx_v.at[0].at[0]],
                              sb0, add=True)
        d2.wait()
        d3.wait()

    plsc.subcore_barrier()

    @pl.when(s < NS - 1)
    def _():
        pltpu.sync_copy(acc.at[pl.ds(s * RPT, RPT)],
                        parts_hbm.at[c].at[pl.ds(s * RPT, RPT)])

    @pl.when(s == NS - 1)
    def _():
        pltpu.sync_copy(acc.at[pl.ds((NS - 1) * RPT, LASTR)],
                        parts_hbm.at[c].at[pl.ds((NS - 1) * RPT, LASTR)])
        pltpu.sync_copy(zeros_hbm.at[pl.ds(0, NPAD - N)],
                        parts_hbm.at[c].at[pl.ds(N, NPAD - N)])


@functools.cache
def _build_sc_kernel():
    return pl.kernel(
        _sc_agg_body,
        out_type=jax.ShapeDtypeStruct((NC, NPAD, F), jnp.float32),
        mesh=plsc.VectorSubcoreMesh(core_axis_name="c", subcore_axis_name="s",
                                    num_cores=NC, num_subcores=NS),
        scratch_types=(
            [
                pltpu.VMEM((2, 2, CHUNK), jnp.int32),
                pltpu.VMEM((CHUNK, F), jnp.float32),
                pltpu.VMEM((CHUNK, F), jnp.float32),
                pltpu.VMEM((CHUNK // 2, F), jnp.int32),
                pltpu.VMEM_SHARED((N, F), jnp.float32),
            ]
            + [pltpu.SemaphoreType.DMA] * 6
        ),
    )


def _sc_aggregate(x_pad, idx, m, zeros_hbm):
    return _build_sc_kernel()(x_pad, idx, m, zeros_hbm)


def _node_body(x_ref, a0_ref, a1_ref, t_ref,
               we1, be1, we2, be2,
               wroot, broot, wi1a, wi1b, bi1, wi2, bi2, wj, bj,
               out_ref):
    x = x_ref[:]
    hl = _softsign(we1[:] + be1[:])
    ml = _softsign(jnp.dot(hl, we2[:], preferred_element_type=jnp.float32)
                   + be2[:])
    ml = ml * jnp.where(t_ref[0] >= 1.0, 1.0, 0.0)
    h1 = (a0_ref[0] + a1_ref[0] + 2.0 * x * ml
          + jnp.dot(x, wroot[:], preferred_element_type=jnp.float32)
          + broot[:])
    a = _softsign(jnp.dot(h1, wi1a[:], preferred_element_type=jnp.float32)
                  + jnp.dot(x, wi1b[:], preferred_element_type=jnp.float32)
                  + bi1[:])
    a = _softsign(jnp.dot(a, wi2[:], preferred_element_type=jnp.float32)
                  + bi2[:])
    sm = jax.nn.softmax(a, axis=1)
    out_ref[:] = sm * _softsign(
        jnp.dot(x, wj[:], preferred_element_type=jnp.float32) + bj[:])


def _node_update(x_pad, parts, t,
                 We1, be1, We2, be2,
                 Wroot, broot, Wi1, bi1, Wi2, bi2, Wj, bj, width):
    full = lambda a: pl.BlockSpec(a.shape, lambda i: (0,) * a.ndim)
    smem = pl.BlockSpec(memory_space=pltpu.SMEM)
    blk = lambda w: pl.BlockSpec((BN, w), lambda i: (i, 0))
    wi1a = Wi1[:F]
    wi1b = Wi1[F:]
    return pl.pallas_call(
        _node_body,
        grid=(NBLK,),
        in_specs=[
            blk(F),
            pl.BlockSpec((1, BN, F), lambda i: (0, i, 0)),
            pl.BlockSpec((1, BN, F), lambda i: (1, i, 0)),
            smem,
            full(We1), full(be1), full(We2), full(be2),
            full(Wroot), full(broot), full(wi1a), full(wi1b), full(bi1),
            full(Wi2), full(bi2), full(Wj), full(bj),
        ],
        out_specs=blk(width),
        out_shape=jax.ShapeDtypeStruct((NPAD, width), jnp.float32),
    )(x_pad, parts, parts, t,
      We1, be1, We2, be2,
      Wroot, broot, wi1a, wi1b, bi1, Wi2, bi2, Wj, bj)


def _pool_body(x_ref, h_ref, b_ref,
               wf0, bf0, wf1, bf1, wf2, bf2,
               out_ref, pool):
    i = pl.program_id(0)

    @pl.when(i == 0)
    def _():
        pool[:] = jnp.zeros_like(pool)

    lig = jnp.where(x_ref[:, 14:15] == -1.0, 0.0, h_ref[:])
    seg = b_ref[:][:, None]
    oh = (seg == lax.broadcasted_iota(jnp.int32, (BN, NB), 1)
          ).astype(jnp.float32)
    pool[:] += lax.dot_general(oh, lig, (((0,), (0,)), ((), ())),
                               preferred_element_type=jnp.float32)

    @pl.when(i == NBLK - 1)
    def _():
        h = jax.nn.relu(jnp.dot(pool[:], wf0[:],
                                preferred_element_type=jnp.float32) + bf0[:])
        h = jax.nn.relu(jnp.dot(h, wf1[:],
                                preferred_element_type=jnp.float32) + bf1[:])
        out_ref[:] = jnp.dot(h, wf2[:],
                             preferred_element_type=jnp.float32) + bf2[:]


def _pool_mlp(x_pad, ncov, batch_pad, Wf0, bf0, Wf1, bf1, Wf2, bf2):
    full = lambda a: pl.BlockSpec(a.shape, lambda i: (0,) * a.ndim)
    return pl.pallas_call(
        _pool_body,
        grid=(NBLK,),
        in_specs=[
            pl.BlockSpec((BN, F), lambda i: (i, 0)),
            pl.BlockSpec((BN, GN), lambda i: (i, 0)),
            pl.BlockSpec((BN,), lambda i: (i,)),
            full(Wf0), full(bf0), full(Wf1), full(bf1), full(Wf2), full(bf2),
        ],
        out_specs=pl.BlockSpec((NB, 1), lambda i: (0, 0)),
        out_shape=jax.ShapeDtypeStruct((NB, 1), jnp.float32),
        scratch_shapes=[pltpu.VMEM((NB, NB), jnp.float32)],
    )(x_pad, ncov, batch_pad, Wf0, bf0, Wf1, bf1, Wf2, bf2)


def kernel(x, edge_index, edge_attr, batch, t_cov, t_ncov,
           We1_c, be1_c, We2_c, be2_c, Wroot_c, broot_c,
           Wi1_c, bi1_c, Wi2_c, bi2_c, Wj_c, bj_c,
           We1_n, be1_n, We2_n, be2_n, Wroot_n, broot_n,
           Wi1_n, bi1_n, Wi2_n, bi2_n, Wj_n, bj_n,
           Wf0, bf0, Wf1, bf1, Wf2, bf2):
    epad = EPAD - E
    ea_pad = jnp.concatenate([edge_attr,
                              jnp.full((epad,), 2.0, jnp.float32)])
    src_pad = jnp.concatenate([edge_index[0],
                               jnp.zeros((epad,), jnp.int32)])
    dst_pad = jnp.concatenate([edge_index[1],
                               jnp.zeros((epad,), jnp.int32)])
    x_pad = jnp.concatenate([x, jnp.zeros((NPAD - N, F), jnp.float32)])
    batch_pad = jnp.concatenate([batch,
                                 jnp.full((NPAD - N,), NB, jnp.int32)])
    zeros_hbm = jnp.zeros((RPT, F), jnp.float32)

    m_c, m_n = _edge_weights(ea_pad, t_cov, t_ncov, jnp.asarray(_PERM),
                             We1_c, be1_c, We2_c, be2_c,
                             We1_n, be1_n, We2_n, be2_n)
    idx = jnp.stack([src_pad.reshape(TOTCHUNK, CHUNK),
                     dst_pad.reshape(TOTCHUNK, CHUNK)], axis=1)
    m_c = lax.bitcast_convert_type(
        m_c.reshape(TOTCHUNK, CHUNK, F // 2, 2), jnp.int32
    ).reshape(TOTCHUNK, CHUNK // 2, F)
    m_n = lax.bitcast_convert_type(
        m_n.reshape(TOTCHUNK, CHUNK, F // 2, 2), jnp.int32
    ).reshape(TOTCHUNK, CHUNK // 2, F)

    parts_c = _sc_aggregate(x_pad, idx, m_c, zeros_hbm)
    cov = _node_update(x_pad, parts_c, t_cov,
                       We1_c, be1_c, We2_c, be2_c,
                       Wroot_c, broot_c, Wi1_c, bi1_c,
                       Wi2_c, bi2_c, Wj_c, bj_c, F)

    parts_n = _sc_aggregate(cov, idx, m_n, zeros_hbm)
    ncov = _node_update(cov, parts_n, t_ncov,
                        We1_n, be1_n, We2_n, be2_n,
                        Wroot_n, broot_n, Wi1_n, bi1_n,
                        Wi2_n, bi2_n, Wj_n, bj_n, GN)

    return _pool_mlp(x_pad, ncov, batch_pad, Wf0, bf0, Wf1, bf1, Wf2, bf2)


# speedup vs baseline: 1.8882x; 1.8114x over previous
import functools

import jax
import jax.numpy as jnp
import numpy as np
from jax import lax
from jax.experimental import pallas as pl
from jax.experimental.pallas import tpu as pltpu
from jax.experimental.pallas import tpu_sc as plsc

N = 10000
E = 320000
F = 128
GN = 64
NB = 64

NC = 2
NS = 16
NW = NC * NS
CHUNK = 128
EPT = 10240
NCHUNK = EPT // CHUNK
TOTCHUNK = NW * NCHUNK
C0CH = 110
C1CH = 50
EPAD = NW * EPT
NPAD = 10240
BN = 512
NBLK = NPAD // BN
RPT = 640
LASTR = N - (NS - 1) * RPT
BE = 1024


def _softsign(v):
    return v / (1.0 + jnp.abs(v))


def _edge_body(ea_ref, tc_ref, tn_ref,
               we1c, be1c, we2c, be2c,
               we1n, be1n, we2n, be2n,
               mc_ref, mn_ref):
    ea = ea_ref[:]
    eacol = ea[:, None]
    hc = _softsign(eacol * we1c[:] + be1c[:])
    mc = _softsign(jnp.dot(hc, we2c[:], preferred_element_type=jnp.float32)
                   + be2c[:])
    hn = _softsign(eacol * we1n[:] + be1n[:])
    mn = _softsign(jnp.dot(hn, we2n[:], preferred_element_type=jnp.float32)
                   + be2n[:])
    maskc = (ea <= tc_ref[0]).astype(jnp.float32)[:, None]
    maskn = (ea <= tn_ref[0]).astype(jnp.float32)[:, None]
    mc_ref[:] = mc * maskc
    mn_ref[:] = mn * maskn


def _edge_weights(ea_pad, t_cov, t_ncov,
                  We1_c, be1_c, We2_c, be2_c,
                  We1_n, be1_n, We2_n, be2_n):
    full = lambda a: pl.BlockSpec(a.shape, lambda i: (0,) * a.ndim)
    smem = pl.BlockSpec(memory_space=pltpu.SMEM)
    return pl.pallas_call(
        _edge_body,
        grid=(EPAD // BE,),
        in_specs=[
            pl.BlockSpec((BE,), lambda i: (i,)),
            smem, smem,
            full(We1_c), full(be1_c), full(We2_c), full(be2_c),
            full(We1_n), full(be1_n), full(We2_n), full(be2_n),
        ],
        out_specs=[
            pl.BlockSpec((BE, F), lambda i: (i, 0)),
            pl.BlockSpec((BE, F), lambda i: (i, 0)),
        ],
        out_shape=[
            jax.ShapeDtypeStruct((EPAD, F), jnp.float32),
            jax.ShapeDtypeStruct((EPAD, F), jnp.float32),
        ],
    )(ea_pad, t_cov, t_ncov,
      We1_c, be1_c, We2_c, be2_c,
      We1_n, be1_n, We2_n, be2_n)


def _sc_agg_body(x_hbm, idx_hbm, m_hbm, zeros_hbm, parts_hbm,
                 idx_v, xs_v, xd_v, mm_v, acc,
                 gx0, gd0, sf0, sb0, gm0, gm1):
    c = lax.axis_index("c")
    s = lax.axis_index("s")

    @pl.loop(0, CHUNK)
    def _zrow(e):
        for v in range(F // 16):
            xs_v[e, pl.ds(v * 16, 16)] = jnp.zeros((16,), jnp.float32)

    @pl.when(s < NS - 1)
    def _():
        for k in range(RPT // CHUNK):
            pltpu.sync_copy(xs_v, acc.at[pl.ds(s * RPT + k * CHUNK, CHUNK)])

    @pl.when(s == NS - 1)
    def _():
        base = (NS - 1) * RPT
        for k in range(LASTR // CHUNK):
            pltpu.sync_copy(xs_v, acc.at[pl.ds(base + k * CHUNK, CHUNK)])
        rem = LASTR % CHUNK
        if rem:
            pltpu.sync_copy(xs_v.at[pl.ds(0, rem)],
                            acc.at[pl.ds(base + LASTR - rem, rem)])

    plsc.subcore_barrier()

    cbase = jnp.where(c == 0, s * C0CH, NS * C0CH + s * C1CH)
    ccnt = jnp.where(c == 0, C0CH, C1CH)
    @pl.loop(0, ccnt)
    def _chunk(jj):
        j = cbase + jj
        pltpu.sync_copy(idx_hbm.at[j], idx_v.at[0])
        d0 = pltpu.async_copy(x_hbm.at[idx_v.at[0].at[0]], xs_v, gx0)
        d1 = pltpu.async_copy(x_hbm.at[idx_v.at[0].at[1]], xd_v, gd0)
        pltpu.sync_copy(m_hbm.at[j], mm_v)
        d0.wait()
        d1.wait()

        @pl.loop(0, CHUNK)
        def _row(e):
            for v in range(F // 16):
                sl = pl.ds(v * 16, 16)
                mm = mm_v[e, sl]
                xs_v[e, sl] = xs_v[e, sl] * mm
                xd_v[e, sl] = xd_v[e, sl] * mm

        d2 = pltpu.async_copy(xs_v, acc.at[idx_v.at[0].at[1]],
                              sf0, add=True)
        d3 = pltpu.async_copy(xd_v, acc.at[idx_v.at[0].at[0]],
                              sb0, add=True)
        d2.wait()
        d3.wait()

    plsc.subcore_barrier()

    @pl.when(s < NS - 1)
    def _():
        pltpu.sync_copy(acc.at[pl.ds(s * RPT, RPT)],
                        parts_hbm.at[c].at[pl.ds(s * RPT, RPT)])

    @pl.when(s == NS - 1)
    def _():
        pltpu.sync_copy(acc.at[pl.ds((NS - 1) * RPT, LASTR)],
                        parts_hbm.at[c].at[pl.ds((NS - 1) * RPT, LASTR)])
        pltpu.sync_copy(zeros_hbm.at[pl.ds(0, NPAD - N)],
                        parts_hbm.at[c].at[pl.ds(N, NPAD - N)])


@functools.cache
def _build_sc_kernel():
    return pl.kernel(
        _sc_agg_body,
        out_type=jax.ShapeDtypeStruct((NC, NPAD, F), jnp.float32),
        mesh=plsc.VectorSubcoreMesh(core_axis_name="c", subcore_axis_name="s",
                                    num_cores=NC, num_subcores=NS),
        scratch_types=(
            [
                pltpu.VMEM((2, 2, CHUNK), jnp.int32),
                pltpu.VMEM((CHUNK, F), jnp.float32),
                pltpu.VMEM((CHUNK, F), jnp.float32),
                pltpu.VMEM((CHUNK, F), jnp.float32),
                pltpu.VMEM_SHARED((N, F), jnp.float32),
            ]
            + [pltpu.SemaphoreType.DMA] * 6
        ),
    )


def _sc_aggregate(x_pad, idx, m, zeros_hbm):
    return _build_sc_kernel()(x_pad, idx, m, zeros_hbm)


def _node_body(x_ref, a0_ref, a1_ref, t_ref,
               we1, be1, we2, be2,
               wroot, broot, wi1a, wi1b, bi1, wi2, bi2, wj, bj,
               out_ref):
    x = x_ref[:]
    hl = _softsign(we1[:] + be1[:])
    ml = _softsign(jnp.dot(hl, we2[:], preferred_element_type=jnp.float32)
                   + be2[:])
    ml = ml * jnp.where(t_ref[0] >= 1.0, 1.0, 0.0)
    h1 = (a0_ref[0] + a1_ref[0] + 2.0 * x * ml
          + jnp.dot(x, wroot[:], preferred_element_type=jnp.float32)
          + broot[:])
    a = _softsign(jnp.dot(h1, wi1a[:], preferred_element_type=jnp.float32)
                  + jnp.dot(x, wi1b[:], preferred_element_type=jnp.float32)
                  + bi1[:])
    a = _softsign(jnp.dot(a, wi2[:], preferred_element_type=jnp.float32)
                  + bi2[:])
    sm = jax.nn.softmax(a, axis=1)
    out_ref[:] = sm * _softsign(
        jnp.dot(x, wj[:], preferred_element_type=jnp.float32) + bj[:])


def _node_update(x_pad, parts, t,
                 We1, be1, We2, be2,
                 Wroot, broot, Wi1, bi1, Wi2, bi2, Wj, bj, width):
    full = lambda a: pl.BlockSpec(a.shape, lambda i: (0,) * a.ndim)
    smem = pl.BlockSpec(memory_space=pltpu.SMEM)
    blk = lambda w: pl.BlockSpec((BN, w), lambda i: (i, 0))
    wi1a = Wi1[:F]
    wi1b = Wi1[F:]
    return pl.pallas_call(
        _node_body,
        grid=(NBLK,),
        in_specs=[
            blk(F),
            pl.BlockSpec((1, BN, F), lambda i: (0, i, 0)),
            pl.BlockSpec((1, BN, F), lambda i: (1, i, 0)),
            smem,
            full(We1), full(be1), full(We2), full(be2),
            full(Wroot), full(broot), full(wi1a), full(wi1b), full(bi1),
            full(Wi2), full(bi2), full(Wj), full(bj),
        ],
        out_specs=blk(width),
        out_shape=jax.ShapeDtypeStruct((NPAD, width), jnp.float32),
    )(x_pad, parts, parts, t,
      We1, be1, We2, be2,
      Wroot, broot, wi1a, wi1b, bi1, Wi2, bi2, Wj, bj)


def _pool_body(x_ref, h_ref, b_ref,
               wf0, bf0, wf1, bf1, wf2, bf2,
               out_ref, pool):
    i = pl.program_id(0)

    @pl.when(i == 0)
    def _():
        pool[:] = jnp.zeros_like(pool)

    lig = jnp.where(x_ref[:, 14:15] == -1.0, 0.0, h_ref[:])
    seg = b_ref[:][:, None]
    oh = (seg == lax.broadcasted_iota(jnp.int32, (BN, NB), 1)
          ).astype(jnp.float32)
    pool[:] += lax.dot_general(oh, lig, (((0,), (0,)), ((), ())),
                               preferred_element_type=jnp.float32)

    @pl.when(i == NBLK - 1)
    def _():
        h = jax.nn.relu(jnp.dot(pool[:], wf0[:],
                                preferred_element_type=jnp.float32) + bf0[:])
        h = jax.nn.relu(jnp.dot(h, wf1[:],
                                preferred_element_type=jnp.float32) + bf1[:])
        out_ref[:] = jnp.dot(h, wf2[:],
                             preferred_element_type=jnp.float32) + bf2[:]


def _pool_mlp(x_pad, ncov, batch_pad, Wf0, bf0, Wf1, bf1, Wf2, bf2):
    full = lambda a: pl.BlockSpec(a.shape, lambda i: (0,) * a.ndim)
    return pl.pallas_call(
        _pool_body,
        grid=(NBLK,),
        in_specs=[
            pl.BlockSpec((BN, F), lambda i: (i, 0)),
            pl.BlockSpec((BN, GN), lambda i: (i, 0)),
            pl.BlockSpec((BN,), lambda i: (i,)),
            full(Wf0), full(bf0), full(Wf1), full(bf1), full(Wf2), full(bf2),
        ],
        out_specs=pl.BlockSpec((NB, 1), lambda i: (0, 0)),
        out_shape=jax.ShapeDtypeStruct((NB, 1), jnp.float32),
        scratch_shapes=[pltpu.VMEM((NB, NB), jnp.float32)],
    )(x_pad, ncov, batch_pad, Wf0, bf0, Wf1, bf1, Wf2, bf2)


def kernel(x, edge_index, edge_attr, batch, t_cov, t_ncov,
           We1_c, be1_c, We2_c, be2_c, Wroot_c, broot_c,
           Wi1_c, bi1_c, Wi2_c, bi2_c, Wj_c, bj_c,
           We1_n, be1_n, We2_n, be2_n, Wroot_n, broot_n,
           Wi1_n, bi1_n, Wi2_n, bi2_n, Wj_n, bj_n,
           Wf0, bf0, Wf1, bf1, Wf2, bf2):
    epad = EPAD - E
    ea_pad = jnp.concatenate([edge_attr,
                              jnp.full((epad,), 2.0, jnp.float32)])
    src_pad = jnp.concatenate([edge_index[0],
                               jnp.zeros((epad,), jnp.int32)])
    dst_pad = jnp.concatenate([edge_index[1],
                               jnp.zeros((epad,), jnp.int32)])
    x_pad = jnp.concatenate([x, jnp.zeros((NPAD - N, F), jnp.float32)])
    batch_pad = jnp.concatenate([batch,
                                 jnp.full((NPAD - N,), NB, jnp.int32)])
    zeros_hbm = jnp.zeros((RPT, F), jnp.float32)

    m_c, m_n = _edge_weights(ea_pad, t_cov, t_ncov,
                             We1_c, be1_c, We2_c, be2_c,
                             We1_n, be1_n, We2_n, be2_n)
    idx = jnp.stack([src_pad.reshape(TOTCHUNK, CHUNK),
                     dst_pad.reshape(TOTCHUNK, CHUNK)], axis=1)
    m_c = m_c.reshape(TOTCHUNK, CHUNK, F)
    m_n = m_n.reshape(TOTCHUNK, CHUNK, F)

    parts_c = _sc_aggregate(x_pad, idx, m_c, zeros_hbm)
    cov = _node_update(x_pad, parts_c, t_cov,
                       We1_c, be1_c, We2_c, be2_c,
                       Wroot_c, broot_c, Wi1_c, bi1_c,
                       Wi2_c, bi2_c, Wj_c, bj_c, F)

    parts_n = _sc_aggregate(cov, idx, m_n, zeros_hbm)
    ncov = _node_update(cov, parts_n, t_ncov,
                        We1_n, be1_n, We2_n, be2_n,
                        Wroot_n, broot_n, Wi1_n, bi1_n,
                        Wi2_n, bi2_n, Wj_n, bj_n, GN)

    return _pool_mlp(x_pad, ncov, batch_pad, Wf0, bf0, Wf1, bf1, Wf2, bf2)


# speedup vs baseline: 1.8894x; 1.0007x over previous
import functools

import jax
import jax.numpy as jnp
import numpy as np
from jax import lax
from jax.experimental import pallas as pl
from jax.experimental.pallas import tpu as pltpu
from jax.experimental.pallas import tpu_sc as plsc

N = 10000
E = 320000
F = 128
GN = 64
NB = 64

NC = 2
NS = 16
NW = NC * NS
CHUNK = 128
EPT = 10240
NCHUNK = EPT // CHUNK
TOTCHUNK = NW * NCHUNK
C0CH = 110
C1CH = 50
EPAD = NW * EPT
NPAD = 10240
BN = 512
NBLK = NPAD // BN
RPT = 640
LASTR = N - (NS - 1) * RPT
BE = 1024


def _softsign(v):
    return v / (1.0 + jnp.abs(v))


def _edge_body(ea_ref, tc_ref, tn_ref,
               we1c, be1c, we2c, be2c,
               we1n, be1n, we2n, be2n,
               mc_ref, mn_ref):
    ea = ea_ref[:]
    eacol = ea[:, None]
    hc = _softsign(eacol * we1c[:] + be1c[:])
    mc = _softsign(jnp.dot(hc, we2c[:], preferred_element_type=jnp.float32)
                   + be2c[:])
    hn = _softsign(eacol * we1n[:] + be1n[:])
    mn = _softsign(jnp.dot(hn, we2n[:], preferred_element_type=jnp.float32)
                   + be2n[:])
    maskc = (ea <= tc_ref[0]).astype(jnp.float32)[:, None]
    maskn = (ea <= tn_ref[0]).astype(jnp.float32)[:, None]
    mc_ref[:] = mc * maskc
    mn_ref[:] = mn * maskn


def _edge_weights(ea_pad, t_cov, t_ncov,
                  We1_c, be1_c, We2_c, be2_c,
                  We1_n, be1_n, We2_n, be2_n):
    full = lambda a: pl.BlockSpec(a.shape, lambda i: (0,) * a.ndim)
    smem = pl.BlockSpec(memory_space=pltpu.SMEM)
    return pl.pallas_call(
        _edge_body,
        grid=(EPAD // BE,),
        in_specs=[
            pl.BlockSpec((BE,), lambda i: (i,)),
            smem, smem,
            full(We1_c), full(be1_c), full(We2_c), full(be2_c),
            full(We1_n), full(be1_n), full(We2_n), full(be2_n),
        ],
        out_specs=[
            pl.BlockSpec((BE, F), lambda i: (i, 0)),
            pl.BlockSpec((BE, F), lambda i: (i, 0)),
        ],
        out_shape=[
            jax.ShapeDtypeStruct((EPAD, F), jnp.float32),
            jax.ShapeDtypeStruct((EPAD, F), jnp.float32),
        ],
    )(ea_pad, t_cov, t_ncov,
      We1_c, be1_c, We2_c, be2_c,
      We1_n, be1_n, We2_n, be2_n)


def _sc_agg_body(x_hbm, idx_hbm, m_hbm, zeros_hbm, parts_hbm,
                 idx_v, xs_v, xd_v, mm_v, acc,
                 gx0, gd0, sf0, sb0):
    c = lax.axis_index("c")
    s = lax.axis_index("s")

    @pl.loop(0, CHUNK)
    def _zrow(e):
        for v in range(F // 16):
            xs_v[e, pl.ds(v * 16, 16)] = jnp.zeros((16,), jnp.float32)

    @pl.when(s < NS - 1)
    def _():
        for k in range(RPT // CHUNK):
            pltpu.sync_copy(xs_v, acc.at[pl.ds(s * RPT + k * CHUNK, CHUNK)])

    @pl.when(s == NS - 1)
    def _():
        base = (NS - 1) * RPT
        for k in range(LASTR // CHUNK):
            pltpu.sync_copy(xs_v, acc.at[pl.ds(base + k * CHUNK, CHUNK)])
        rem = LASTR % CHUNK
        if rem:
            pltpu.sync_copy(xs_v.at[pl.ds(0, rem)],
                            acc.at[pl.ds(base + LASTR - rem, rem)])

    plsc.subcore_barrier()

    cbase = jnp.where(c == 0, s * C0CH, NS * C0CH + s * C1CH)
    ccnt = jnp.where(c == 0, C0CH, C1CH)
    @pl.loop(0, ccnt)
    def _chunk(jj):
        j = cbase + jj
        pltpu.sync_copy(idx_hbm.at[j], idx_v)
        d0 = pltpu.async_copy(x_hbm.at[idx_v.at[0]], xs_v, gx0)
        d1 = pltpu.async_copy(x_hbm.at[idx_v.at[1]], xd_v, gd0)
        pltpu.sync_copy(m_hbm.at[j], mm_v)
        d0.wait()
        d1.wait()

        @pl.loop(0, CHUNK)
        def _row(e):
            for v in range(F // 16):
                sl = pl.ds(v * 16, 16)
                mm = mm_v[e, sl]
                xs_v[e, sl] = xs_v[e, sl] * mm
                xd_v[e, sl] = xd_v[e, sl] * mm

        d2 = pltpu.async_copy(xs_v, acc.at[idx_v.at[1]], sf0, add=True)
        d3 = pltpu.async_copy(xd_v, acc.at[idx_v.at[0]], sb0, add=True)
        d2.wait()
        d3.wait()

    plsc.subcore_barrier()

    @pl.when(s < NS - 1)
    def _():
        pltpu.sync_copy(acc.at[pl.ds(s * RPT, RPT)],
                        parts_hbm.at[c].at[pl.ds(s * RPT, RPT)])

    @pl.when(s == NS - 1)
    def _():
        pltpu.sync_copy(acc.at[pl.ds((NS - 1) * RPT, LASTR)],
                        parts_hbm.at[c].at[pl.ds((NS - 1) * RPT, LASTR)])
        pltpu.sync_copy(zeros_hbm.at[pl.ds(0, NPAD - N)],
                        parts_hbm.at[c].at[pl.ds(N, NPAD - N)])


@functools.cache
def _build_sc_kernel():
    return pl.kernel(
        _sc_agg_body,
        out_type=jax.ShapeDtypeStruct((NC, NPAD, F), jnp.float32),
        mesh=plsc.VectorSubcoreMesh(core_axis_name="c", subcore_axis_name="s",
                                    num_cores=NC, num_subcores=NS),
        scratch_types=(
            [
                pltpu.VMEM((2, CHUNK), jnp.int32),
                pltpu.VMEM((CHUNK, F), jnp.float32),
                pltpu.VMEM((CHUNK, F), jnp.float32),
                pltpu.VMEM((CHUNK, F), jnp.float32),
                pltpu.VMEM_SHARED((N, F), jnp.float32),
            ]
            + [pltpu.SemaphoreType.DMA] * 4
        ),
    )


def _sc_aggregate(x_pad, idx, m, zeros_hbm):
    return _build_sc_kernel()(x_pad, idx, m, zeros_hbm)


def _node_body(x_ref, a0_ref, a1_ref, t_ref,
               we1, be1, we2, be2,
               wroot, broot, wi1a, wi1b, bi1, wi2, bi2, wj, bj,
               out_ref):
    x = x_ref[:]
    hl = _softsign(we1[:] + be1[:])
    ml = _softsign(jnp.dot(hl, we2[:], preferred_element_type=jnp.float32)
                   + be2[:])
    ml = ml * jnp.where(t_ref[0] >= 1.0, 1.0, 0.0)
    h1 = (a0_ref[0] + a1_ref[0] + 2.0 * x * ml
          + jnp.dot(x, wroot[:], preferred_element_type=jnp.float32)
          + broot[:])
    a = _softsign(jnp.dot(h1, wi1a[:], preferred_element_type=jnp.float32)
                  + jnp.dot(x, wi1b[:], preferred_element_type=jnp.float32)
                  + bi1[:])
    a = _softsign(jnp.dot(a, wi2[:], preferred_element_type=jnp.float32)
                  + bi2[:])
    sm = jax.nn.softmax(a, axis=1)
    out_ref[:] = sm * _softsign(
        jnp.dot(x, wj[:], preferred_element_type=jnp.float32) + bj[:])


def _node_update(x_pad, parts, t,
                 We1, be1, We2, be2,
                 Wroot, broot, Wi1, bi1, Wi2, bi2, Wj, bj, width):
    full = lambda a: pl.BlockSpec(a.shape, lambda i: (0,) * a.ndim)
    smem = pl.BlockSpec(memory_space=pltpu.SMEM)
    blk = lambda w: pl.BlockSpec((BN, w), lambda i: (i, 0))
    wi1a = Wi1[:F]
    wi1b = Wi1[F:]
    return pl.pallas_call(
        _node_body,
        grid=(NBLK,),
        in_specs=[
            blk(F),
            pl.BlockSpec((1, BN, F), lambda i: (0, i, 0)),
            pl.BlockSpec((1, BN, F), lambda i: (1, i, 0)),
            smem,
            full(We1), full(be1), full(We2), full(be2),
            full(Wroot), full(broot), full(wi1a), full(wi1b), full(bi1),
            full(Wi2), full(bi2), full(Wj), full(bj),
        ],
        out_specs=blk(width),
        out_shape=jax.ShapeDtypeStruct((NPAD, width), jnp.float32),
    )(x_pad, parts, parts, t,
      We1, be1, We2, be2,
      Wroot, broot, wi1a, wi1b, bi1, Wi2, bi2, Wj, bj)


def _pool_body(x_ref, h_ref, b_ref,
               wf0, bf0, wf1, bf1, wf2, bf2,
               out_ref, pool):
    i = pl.program_id(0)

    @pl.when(i == 0)
    def _():
        pool[:] = jnp.zeros_like(pool)

    lig = jnp.where(x_ref[:, 14:15] == -1.0, 0.0, h_ref[:])
    seg = b_ref[:][:, None]
    oh = (seg == lax.broadcasted_iota(jnp.int32, (BN, NB), 1)
          ).astype(jnp.float32)
    pool[:] += lax.dot_general(oh, lig, (((0,), (0,)), ((), ())),
                               preferred_element_type=jnp.float32)

    @pl.when(i == NBLK - 1)
    def _():
        h = jax.nn.relu(jnp.dot(pool[:], wf0[:],
                                preferred_element_type=jnp.float32) + bf0[:])
        h = jax.nn.relu(jnp.dot(h, wf1[:],
                                preferred_element_type=jnp.float32) + bf1[:])
        out_ref[:] = jnp.dot(h, wf2[:],
                             preferred_element_type=jnp.float32) + bf2[:]


def _pool_mlp(x_pad, ncov, batch_pad, Wf0, bf0, Wf1, bf1, Wf2, bf2):
    full = lambda a: pl.BlockSpec(a.shape, lambda i: (0,) * a.ndim)
    return pl.pallas_call(
        _pool_body,
        grid=(NBLK,),
        in_specs=[
            pl.BlockSpec((BN, F), lambda i: (i, 0)),
            pl.BlockSpec((BN, GN), lambda i: (i, 0)),
            pl.BlockSpec((BN,), lambda i: (i,)),
            full(Wf0), full(bf0), full(Wf1), full(bf1), full(Wf2), full(bf2),
        ],
        out_specs=pl.BlockSpec((NB, 1), lambda i: (0, 0)),
        out_shape=jax.ShapeDtypeStruct((NB, 1), jnp.float32),
        scratch_shapes=[pltpu.VMEM((NB, NB), jnp.float32)],
    )(x_pad, ncov, batch_pad, Wf0, bf0, Wf1, bf1, Wf2, bf2)


def kernel(x, edge_index, edge_attr, batch, t_cov, t_ncov,
           We1_c, be1_c, We2_c, be2_c, Wroot_c, broot_c,
           Wi1_c, bi1_c, Wi2_c, bi2_c, Wj_c, bj_c,
           We1_n, be1_n, We2_n, be2_n, Wroot_n, broot_n,
           Wi1_n, bi1_n, Wi2_n, bi2_n, Wj_n, bj_n,
           Wf0, bf0, Wf1, bf1, Wf2, bf2):
    epad = EPAD - E
    ea_pad = jnp.concatenate([edge_attr,
                              jnp.full((epad,), 2.0, jnp.float32)])
    src_pad = jnp.concatenate([edge_index[0],
                               jnp.zeros((epad,), jnp.int32)])
    dst_pad = jnp.concatenate([edge_index[1],
                               jnp.zeros((epad,), jnp.int32)])
    x_pad = jnp.concatenate([x, jnp.zeros((NPAD - N, F), jnp.float32)])
    batch_pad = jnp.concatenate([batch,
                                 jnp.full((NPAD - N,), NB, jnp.int32)])
    zeros_hbm = jnp.zeros((RPT, F), jnp.float32)

    m_c, m_n = _edge_weights(ea_pad, t_cov, t_ncov,
                             We1_c, be1_c, We2_c, be2_c,
                             We1_n, be1_n, We2_n, be2_n)
    idx = jnp.stack([src_pad.reshape(TOTCHUNK, CHUNK),
                     dst_pad.reshape(TOTCHUNK, CHUNK)], axis=1)
    m_c = m_c.reshape(TOTCHUNK, CHUNK, F)
    m_n = m_n.reshape(TOTCHUNK, CHUNK, F)

    parts_c = _sc_aggregate(x_pad, idx, m_c, zeros_hbm)
    cov = _node_update(x_pad, parts_c, t_cov,
                       We1_c, be1_c, We2_c, be2_c,
                       Wroot_c, broot_c, Wi1_c, bi1_c,
                       Wi2_c, bi2_c, Wj_c, bj_c, F)

    parts_n = _sc_aggregate(cov, idx, m_n, zeros_hbm)
    ncov = _node_update(cov, parts_n, t_ncov,
                        We1_n, be1_n, We2_n, be2_n,
                        Wroot_n, broot_n, Wi1_n, bi1_n,
                        Wi2_n, bi2_n, Wj_n, bj_n, GN)

    return _pool_mlp(x_pad, ncov, batch_pad, Wf0, bf0, Wf1, bf1, Wf2, bf2)
